# baseline (device time: 36272 ns/iter reference)
import jax
import jax.numpy as jnp
from jax import lax
from jax.experimental import pallas as pl
from jax.experimental.pallas import tpu as pltpu

N_DEV = 4
N_LAYERS = 3
N_COL = 4

def _slot(stage, hf, col):
    return (stage * 2 + hf) * N_COL + col

N_SLOTS = 2 * 2 * N_COL


def kernel(x, Win0, Wout0, Win1, Wout1, Win2, Wout2):
    b, _ = x.shape
    h_dim = Win0.shape[1]
    half = b // 2
    cw = h_dim // N_COL

    def body(x_ref, win0_ref, wout0_ref, win1_ref, wout1_ref, win2_ref,
             wout2_ref, out_ref, acc_ref, xbuf_ref,
             tx1_buf, tx2_buf, rb1_buf, rb2_buf, send_sems, recv_sems):
        my = lax.axis_index("i")
        pa = my ^ 1
        pb = 3 - my
        rows = (pl.ds(0, half), pl.ds(half, half))
        p1 = (pa, pb)
        p2 = (pb, pa)

        barrier_sem = pltpu.get_barrier_semaphore()
        for nbr in (pa, pb):
            pl.semaphore_signal(
                barrier_sem, inc=1,
                device_id=(nbr,), device_id_type=pl.DeviceIdType.MESH,
            )
        pl.semaphore_wait(barrier_sem, 2)

        def exch(stage, hf, col, src_ref, dst_ref, peer):
            sl = _slot(stage, hf, col)
            rdma = pltpu.make_async_remote_copy(
                src_ref=src_ref,
                dst_ref=dst_ref,
                send_sem=send_sems.at[sl],
                recv_sem=recv_sems.at[sl],
                device_id=(peer,),
                device_id_type=pl.DeviceIdType.MESH,
            )
            rdma.start()
            return rdma

        win_refs = [win0_ref, win1_ref, win2_ref]
        wout_refs = [wout0_ref, wout1_ref, wout2_ref]

        for layer in range(N_LAYERS):
            xsrc = x_ref if layer == 0 else xbuf_ref
            xdst = out_ref if layer == N_LAYERS - 1 else xbuf_ref
            win = win_refs[layer]
            wout = wout_refs[layer]

            r1 = {}
            for col in range(N_COL):
                cols = pl.ds(col * cw, cw)
                for hf in range(2):
                    ptile = jnp.dot(
                        xsrc[rows[hf], :], win[:, cols],
                        preferred_element_type=jnp.float32,
                    )
                    acc_ref[rows[hf], cols] = ptile
                    tx1_buf[hf, :, cols] = ptile.astype(jnp.bfloat16)
                    r1[hf, col] = exch(
                        0, hf, col,
                        tx1_buf.at[hf, :, cols], rb1_buf.at[hf, :, cols],
                        p1[hf],
                    )

            r2 = {}
            for col in range(N_COL):
                cols = pl.ds(col * cw, cw)
                for hf in range(2):
                    r1[hf, col].wait_recv()
                    summed = acc_ref[rows[hf], cols] + rb1_buf[
                        hf, :, cols
                    ].astype(jnp.float32)
                    acc_ref[rows[hf], cols] = summed
                    tx2_buf[hf, :, cols] = summed.astype(jnp.bfloat16)
                    r2[hf, col] = exch(
                        1, hf, col,
                        tx2_buf.at[hf, :, cols], rb2_buf.at[hf, :, cols],
                        p2[hf],
                    )

            for col in range(N_COL):
                cols = pl.ds(col * cw, cw)
                for hf in range(2):
                    r2[hf, col].wait_recv()
                    htile = jnp.maximum(
                        acc_ref[rows[hf], cols] + rb2_buf[hf, :, cols].astype(
                            jnp.float32
                        ),
                        0.0,
                    )
                    contrib = jnp.dot(
                        htile, wout[cols, :],
                        preferred_element_type=jnp.float32,
                    )
                    if col == 0:
                        xdst[rows[hf], :] = contrib
                    else:
                        xdst[rows[hf], :] = xdst[rows[hf], :] + contrib

            for r in list(r1.values()) + list(r2.values()):
                r.wait_send()

    return pl.pallas_call(
        body,
        out_shape=jax.ShapeDtypeStruct(x.shape, jnp.float32),
        in_specs=[pl.BlockSpec(memory_space=pltpu.VMEM)] * 7,
        out_specs=pl.BlockSpec(memory_space=pltpu.VMEM),
        scratch_shapes=[
            pltpu.VMEM((b, h_dim), jnp.float32),
            pltpu.VMEM(x.shape, jnp.float32),
            pltpu.VMEM((2, half, h_dim), jnp.bfloat16),
            pltpu.VMEM((2, half, h_dim), jnp.bfloat16),
            pltpu.VMEM((2, half, h_dim), jnp.bfloat16),
            pltpu.VMEM((2, half, h_dim), jnp.bfloat16),
            pltpu.SemaphoreType.DMA((N_SLOTS,)),
            pltpu.SemaphoreType.DMA((N_SLOTS,)),
        ],
        compiler_params=pltpu.CompilerParams(collective_id=0),
    )(x, Win0, Wout0, Win1, Wout1, Win2, Wout2)
